# baseline (speedup 1.0000x reference)
import functools
import math

import jax
import jax.numpy as jnp
from jax import lax
from jax.experimental import pallas as pl
from jax.experimental.pallas import tpu as pltpu
from jax.experimental.pallas import tpu_sc as plsc

B = 1024
SEQ = 200
D = 128
SCALE = math.sqrt(float(D))

NW = 32
SO = 4
NOCT = SEQ // SO
NSPAN = 3
BQ = 64
NBB = B // BQ
UNITS = NOCT * NBB
UNITS_PER_W = UNITS // NW
ROWS_U = BQ * SO
IDX_PER_W = UNITS_PER_W * ROWS_U

NBUF = 3
NTRIPLE = UNITS_PER_W // NBUF

_mesh = plsc.VectorSubcoreMesh(core_axis_name="c", subcore_axis_name="s")


@functools.partial(
    pl.kernel,
    mesh=_mesh,
    out_type=jax.ShapeDtypeStruct((B, NOCT, SO, D), jnp.float32),
    scratch_types=[
        pltpu.VMEM((24, D), jnp.float32),
        pltpu.VMEM((NSPAN * SO * B,), jnp.int32),
        pltpu.VMEM((IDX_PER_W,), jnp.int32),
        pltpu.VMEM((BQ, SO, D), jnp.float32),
        pltpu.VMEM((BQ, SO, D), jnp.float32),
        pltpu.VMEM((BQ, SO, D), jnp.float32),
        pltpu.SemaphoreType.DMA,
        pltpu.SemaphoreType.DMA,
        pltpu.SemaphoreType.DMA,
        pltpu.SemaphoreType.DMA,
        pltpu.SemaphoreType.DMA,
        pltpu.SemaphoreType.DMA,
    ],
)
def _embed(idxg_hbm, idxm_hbm, table_hbm, pos_hbm, out_hbm, pos_v, idxg_all,
           idx_all, rows0, rows1, rows2, g0, g1, g2, o0, o1, o2):
    wid = lax.axis_index("s") * 2 + lax.axis_index("c")
    u0 = wid * UNITS_PER_W
    o_min = u0 // NBB
    o_base = jnp.minimum(o_min, NOCT - NSPAN)
    rows = (rows0, rows1, rows2)
    gsem = (g0, g1, g2)
    osem = (o0, o1, o2)

    pstart = o_base * SO - lax.rem(o_base * SO, 8)
    pstart = pl.multiple_of(jnp.minimum(pstart, SEQ - 24), 8)
    pltpu.sync_copy(pos_hbm.at[pl.ds(pstart, 24)], pos_v)
    pltpu.sync_copy(idxm_hbm.at[pl.ds(u0 * ROWS_U, IDX_PER_W)], idx_all)
    pltpu.sync_copy(idxg_hbm.at[pl.ds(o_base * SO * B, NSPAN * SO * B)], idxg_all)

    def gather_copies(p, k):
        u = u0 + k
        o = u // NBB
        qb = lax.rem(u, NBB)
        cps = []
        for h in range(SO):
            off = ((o - o_base) * SO + h) * B + qb * BQ
            cps.append(pltpu.make_async_copy(
                table_hbm.at[idxg_all.at[pl.ds(off, BQ)]],
                rows[p].at[:, h],
                gsem[p],
            ))
        return cps

    def start_gather(p, k):
        for cp in gather_copies(p, k):
            cp.start()

    def wait_gather(p, k):
        for cp in gather_copies(p, k):
            cp.wait()

    def out_copy(p, k):
        u = u0 + k
        o = u // NBB
        qb = lax.rem(u, NBB)
        return pltpu.make_async_copy(
            rows[p],
            out_hbm.at[pl.ds(qb * BQ, BQ), o],
            osem[p],
        )

    def compute(p, k):
        u = u0 + k
        o = u // NBB
        prow = o * SO - pstart

        pv = [[pos_v[prow + s_loc, pl.ds(j * 16, 16)] for j in range(8)]
              for s_loc in range(SO)]

        def group_body(g, c):
            idx16 = idx_all[pl.ds(k * ROWS_U + g * 16, 16)]
            m16 = jnp.where(idx16 == 0, jnp.float32(0.0), jnp.float32(1.0))
            for r in range(16):
                b_loc = g * 4 + r // 4
                s_loc = r % 4
                m = m16[r]
                for j in range(8):
                    sl = pl.ds(j * 16, 16)
                    v = rows[p][b_loc, s_loc, sl]
                    rows[p][b_loc, s_loc, sl] = \
                        (v * SCALE + pv[s_loc][j]) * m
            return c

        lax.fori_loop(0, ROWS_U // 16, group_body, 0)

    start_gather(0, 0)
    start_gather(1, 1)

    def triple_body(t, c):
        k0 = t * 3
        wait_gather(0, k0)
        compute(0, k0)
        out_copy(0, k0).start()
        wait_gather(1, k0 + 1)

        @pl.when(t > 0)
        def _():
            out_copy(2, k0 - 1).wait()

        start_gather(2, k0 + 2)
        compute(1, k0 + 1)
        out_copy(1, k0 + 1).start()
        out_copy(0, k0).wait()
        start_gather(0, k0 + 3)
        wait_gather(2, k0 + 2)
        compute(2, k0 + 2)
        out_copy(2, k0 + 2).start()
        out_copy(1, k0 + 1).wait()

        @pl.when(t + 1 < NTRIPLE)
        def _():
            start_gather(1, k0 + 4)

        return c

    lax.fori_loop(0, NTRIPLE, triple_body, 0)

    k_epi = NBUF * NTRIPLE
    wait_gather(0, k_epi)
    compute(0, k_epi)
    out_copy(0, k_epi).start()
    out_copy(2, k_epi - 1).wait()
    out_copy(0, k_epi).wait()


def kernel(inputs, token_table, pos_table):
    idxg = inputs.T.reshape(-1)
    idxm = inputs.reshape(B, NOCT, SO).transpose(1, 0, 2).reshape(-1)
    return _embed(idxg, idxm, token_table, pos_table).reshape(B, SEQ, D)
